# baseline (speedup 1.0000x reference)
import jax
import jax.numpy as jnp
from jax import lax
from jax.experimental import pallas as pl
from jax.experimental.pallas import tpu as pltpu

SEQ_BLOCK = 2048


def _add_kernel(x_ref, w_hbm, o_ref, w_vmem, sems):
    i = pl.program_id(0)
    b = pl.program_id(1)
    nb = pl.num_programs(0)
    batch = pl.num_programs(1)
    quarter = SEQ_BLOCK // batch
    cur = lax.rem(i, 2)
    nxt = lax.rem(i + 1, 2)

    @pl.when(jnp.logical_and(i == 0, b == 0))
    def _prime():
        for q in range(batch):
            pltpu.make_async_copy(
                w_hbm.at[pl.ds(q * quarter, quarter), :],
                w_vmem.at[0, pl.ds(q * quarter, quarter), :],
                sems.at[0, q],
            ).start()

    @pl.when(i + 1 < nb)
    def _prefetch():
        pltpu.make_async_copy(
            w_hbm.at[pl.ds((i + 1) * SEQ_BLOCK + b * quarter, quarter), :],
            w_vmem.at[nxt, pl.ds(b * quarter, quarter), :],
            sems.at[nxt, b],
        ).start()

    @pl.when(b == 0)
    def _wait():
        for q in range(batch):
            pltpu.make_async_copy(
                w_hbm.at[pl.ds(q * quarter, quarter), :],
                w_vmem.at[cur, pl.ds(q * quarter, quarter), :],
                sems.at[cur, q],
            ).wait()

    o_ref[...] = x_ref[...] + w_vmem[cur][None, :, :]


def kernel(x, embedding_weight):
    batch, seq_len, hidden = x.shape
    num_blocks = seq_len // SEQ_BLOCK

    return pl.pallas_call(
        _add_kernel,
        grid=(num_blocks, batch),
        in_specs=[
            pl.BlockSpec((1, SEQ_BLOCK, hidden), lambda i, b: (b, i, 0)),
            pl.BlockSpec(memory_space=pltpu.MemorySpace.HBM),
        ],
        out_specs=pl.BlockSpec((1, SEQ_BLOCK, hidden), lambda i, b: (b, i, 0)),
        out_shape=jax.ShapeDtypeStruct(x.shape, x.dtype),
        scratch_shapes=[
            pltpu.VMEM((2, SEQ_BLOCK, hidden), jnp.float32),
            pltpu.SemaphoreType.DMA((2, 4)),
        ],
    )(x, embedding_weight)


# speedup vs baseline: 1.0147x; 1.0147x over previous
import jax
import jax.numpy as jnp
from jax import lax
from jax.experimental import pallas as pl
from jax.experimental.pallas import tpu as pltpu

SEQ_BLOCK = 2048


def _add_kernel(x_ref, w_hbm, o_ref, w_vmem, sems):
    i = pl.program_id(0)
    b = pl.program_id(1)
    nb = pl.num_programs(0)
    batch = pl.num_programs(1)
    quarter = SEQ_BLOCK // batch
    cur = lax.rem(i, 2)
    nxt = lax.rem(i + 1, 2)

    @pl.when(jnp.logical_and(i == 0, b == 0))
    def _prime():
        pltpu.make_async_copy(
            w_hbm.at[pl.ds(0, SEQ_BLOCK), :],
            w_vmem.at[0],
            sems.at[0],
        ).start()

    @pl.when(jnp.logical_and(i + 1 < nb, b == 0))
    def _prefetch():
        pltpu.make_async_copy(
            w_hbm.at[pl.ds((i + 1) * SEQ_BLOCK, SEQ_BLOCK), :],
            w_vmem.at[nxt],
            sems.at[nxt],
        ).start()

    @pl.when(b == 0)
    def _wait():
        pltpu.make_async_copy(
            w_hbm.at[pl.ds(0, SEQ_BLOCK), :],
            w_vmem.at[cur],
            sems.at[cur],
        ).wait()

    o_ref[...] = x_ref[...] + w_vmem[cur][None, :, :]


def kernel(x, embedding_weight):
    batch, seq_len, hidden = x.shape
    num_blocks = seq_len // SEQ_BLOCK

    return pl.pallas_call(
        _add_kernel,
        grid=(num_blocks, batch),
        in_specs=[
            pl.BlockSpec((1, SEQ_BLOCK, hidden), lambda i, b: (b, i, 0)),
            pl.BlockSpec(memory_space=pltpu.MemorySpace.HBM),
        ],
        out_specs=pl.BlockSpec((1, SEQ_BLOCK, hidden), lambda i, b: (b, i, 0)),
        out_shape=jax.ShapeDtypeStruct(x.shape, x.dtype),
        scratch_shapes=[
            pltpu.VMEM((2, SEQ_BLOCK, hidden), jnp.float32),
            pltpu.SemaphoreType.DMA((2,)),
        ],
    )(x, embedding_weight)


# speedup vs baseline: 1.0182x; 1.0034x over previous
import jax
import jax.numpy as jnp
from jax import lax
from jax.experimental import pallas as pl
from jax.experimental.pallas import tpu as pltpu

SEQ_BLOCK = 2048
NBUF = 4


def _add_kernel(x_hbm, w_hbm, o_hbm, xbuf, wbuf, xsem, wsem, osem):
    batch = x_hbm.shape[0]
    nb = x_hbm.shape[1] // SEQ_BLOCK
    nsteps = nb * batch

    def x_slice(t):
        i = t // batch
        b = t % batch
        return (b, pl.ds(i * SEQ_BLOCK, SEQ_BLOCK), slice(None))

    pltpu.make_async_copy(
        w_hbm.at[pl.ds(0, SEQ_BLOCK), :], wbuf.at[0], wsem.at[0]
    ).start()
    for t in range(NBUF):
        pltpu.make_async_copy(
            x_hbm.at[x_slice(t)], xbuf.at[t], xsem.at[t]
        ).start()

    def step(t, carry):
        r = t % NBUF
        i = t // batch
        b = t % batch
        cur = i % 2

        @pl.when(jnp.logical_and(b == 0, i + 1 < nb))
        def _prefetch_w():
            pltpu.make_async_copy(
                w_hbm.at[pl.ds((i + 1) * SEQ_BLOCK, SEQ_BLOCK), :],
                wbuf.at[(i + 1) % 2],
                wsem.at[(i + 1) % 2],
            ).start()

        @pl.when(b == 0)
        def _wait_w():
            pltpu.make_async_copy(
                w_hbm.at[pl.ds(0, SEQ_BLOCK), :], wbuf.at[cur], wsem.at[cur]
            ).wait()

        pltpu.make_async_copy(
            x_hbm.at[x_slice(t)], xbuf.at[r], xsem.at[r]
        ).wait()

        xbuf[r] = xbuf[r] + wbuf[cur]

        pltpu.make_async_copy(
            xbuf.at[r], o_hbm.at[x_slice(t)], osem.at[r]
        ).start()

        @pl.when(t + NBUF < nsteps)
        def _next_load():
            pltpu.make_async_copy(
                xbuf.at[r], o_hbm.at[x_slice(t)], osem.at[r]
            ).wait()
            pltpu.make_async_copy(
                x_hbm.at[x_slice(t + NBUF)], xbuf.at[r], xsem.at[r]
            ).start()

        return carry

    lax.fori_loop(0, nsteps, step, 0)

    for t in range(nsteps - NBUF, nsteps):
        pltpu.make_async_copy(
            xbuf.at[t % NBUF], o_hbm.at[x_slice(t)], osem.at[t % NBUF]
        ).wait()


def kernel(x, embedding_weight):
    batch, seq_len, hidden = x.shape

    return pl.pallas_call(
        _add_kernel,
        in_specs=[
            pl.BlockSpec(memory_space=pltpu.MemorySpace.HBM),
            pl.BlockSpec(memory_space=pltpu.MemorySpace.HBM),
        ],
        out_specs=pl.BlockSpec(memory_space=pltpu.MemorySpace.HBM),
        out_shape=jax.ShapeDtypeStruct(x.shape, x.dtype),
        scratch_shapes=[
            pltpu.VMEM((NBUF, SEQ_BLOCK, hidden), jnp.float32),
            pltpu.VMEM((2, SEQ_BLOCK, hidden), jnp.float32),
            pltpu.SemaphoreType.DMA((NBUF,)),
            pltpu.SemaphoreType.DMA((2,)),
            pltpu.SemaphoreType.DMA((NBUF,)),
        ],
    )(x, embedding_weight)
